# Initial kernel scaffold; baseline (speedup 1.0000x reference)
import jax
import jax.numpy as jnp
from jax.experimental import pallas as pl


def kernel(x, edge_index, edge_weight, W1, b1, W2, b2):
    raise NotImplementedError("write your pallas kernel here")


# speedup vs baseline: 10.3019x; 10.3019x over previous
import functools

import jax
import jax.numpy as jnp
from jax import lax
from jax.experimental import pallas as pl
from jax.experimental.pallas import tpu as pltpu
from jax.experimental.pallas import tpu_sc as plsc

N = 10000
E = 320000
NC = 2
NS = 16
NW = NC * NS
EPW = E // NW
CH = 80
NCHUNK = EPW // CH
ROWS_A = 624

_MESH = plsc.VectorSubcoreMesh(core_axis_name="c", subcore_axis_name="s")


def _make_edge(D):
    SL = D // 16

    @functools.partial(
        pl.kernel,
        out_type=jax.ShapeDtypeStruct((NC, N, 128), jnp.float32),
        mesh=_MESH,
        scratch_types=[
            pltpu.VMEM((CH,), jnp.int32),
            pltpu.VMEM((CH,), jnp.int32),
            pltpu.VMEM((CH + 16,), jnp.float32),
            pltpu.VMEM((CH, 128), jnp.float32),
            pltpu.VMEM((48, 128), jnp.float32),
            pltpu.VMEM_SHARED((N, 128), jnp.float32),
            pltpu.SemaphoreType.DMA,
        ],
    )
    def k(y_hbm, src_hbm, dst_hbm, ew_hbm, out_hbm,
          sidx, didx, ewv, rows, zbuf, accum, sem):
        c = lax.axis_index("c")
        s = lax.axis_index("s")
        wid = c * NS + s
        r0 = s * ROWS_A

        def zero_body(i, carry):
            for j in range(8):
                zbuf[i, pl.ds(j * 16, 16)] = jnp.zeros((16,), jnp.float32)
            return carry

        lax.fori_loop(0, 48, zero_body, 0)
        for t in range(ROWS_A // 48):
            pltpu.sync_copy(zbuf, accum.at[pl.ds(r0 + t * 48, 48)])

        @pl.when(s == NS - 1)
        def _():
            pltpu.sync_copy(zbuf.at[pl.ds(0, 16)],
                            accum.at[pl.ds(NS * ROWS_A, 16)])

        plsc.subcore_barrier()

        def chunk_body(ci, carry):
            base = wid * EPW + ci * CH
            pltpu.sync_copy(src_hbm.at[pl.ds(base, CH)], sidx)
            pltpu.sync_copy(dst_hbm.at[pl.ds(base, CH)], didx)
            pltpu.sync_copy(ew_hbm.at[pl.ds(base, CH)], ewv.at[pl.ds(0, CH)])
            pltpu.async_copy(y_hbm.at[sidx], rows, sem).wait()

            def group_body(g, carry2):
                wv = ewv[pl.ds(g * 16, 16)]
                for i in range(16):
                    w = wv[i]
                    e = g * 16 + i
                    for j in range(SL):
                        rows[e, pl.ds(j * 16, 16)] = (
                            rows[e, pl.ds(j * 16, 16)] * w)
                return carry2

            lax.fori_loop(0, CH // 16, group_body, 0)
            pltpu.sync_copy(rows, accum.at[didx], add=True)
            return carry

        lax.fori_loop(0, NCHUNK, chunk_body, 0)
        plsc.subcore_barrier()

        for t in range(ROWS_A // 48):
            pltpu.sync_copy(accum.at[pl.ds(r0 + t * 48, 48)], zbuf)
            pltpu.sync_copy(zbuf, out_hbm.at[c, pl.ds(r0 + t * 48, 48)])

        @pl.when(s == NS - 1)
        def _():
            pltpu.sync_copy(accum.at[pl.ds(NS * ROWS_A, 16)],
                            zbuf.at[pl.ds(0, 16)])
            pltpu.sync_copy(zbuf.at[pl.ds(0, 16)],
                            out_hbm.at[c, pl.ds(NS * ROWS_A, 16)])

    return k


@functools.partial(
    pl.kernel,
    out_type=jax.ShapeDtypeStruct((NC * N,), jnp.float32),
    mesh=_MESH,
    scratch_types=[
        pltpu.VMEM((CH,), jnp.int32),
        pltpu.VMEM((CH,), jnp.float32),
        pltpu.VMEM((48,), jnp.float32),
        pltpu.VMEM_SHARED((N,), jnp.float32),
    ],
)
def _deg_kernel(dst_hbm, ew_hbm, out_hbm, didx, ewv, zbuf, accum):
    c = lax.axis_index("c")
    s = lax.axis_index("s")
    wid = c * NS + s
    r0 = s * ROWS_A
    for j in range(3):
        zbuf[pl.ds(j * 16, 16)] = jnp.zeros((16,), jnp.float32)
    for t in range(ROWS_A // 48):
        pltpu.sync_copy(zbuf, accum.at[pl.ds(r0 + t * 48, 48)])

    @pl.when(s == NS - 1)
    def _():
        pltpu.sync_copy(zbuf.at[pl.ds(0, 16)], accum.at[pl.ds(NS * ROWS_A, 16)])

    plsc.subcore_barrier()

    def chunk_body(ci, carry):
        base = wid * EPW + ci * CH
        pltpu.sync_copy(dst_hbm.at[pl.ds(base, CH)], didx)
        pltpu.sync_copy(ew_hbm.at[pl.ds(base, CH)], ewv)
        pltpu.sync_copy(ewv, accum.at[didx], add=True)
        return carry

    lax.fori_loop(0, NCHUNK, chunk_body, 0)
    plsc.subcore_barrier()

    for t in range(ROWS_A // 48):
        pltpu.sync_copy(accum.at[pl.ds(r0 + t * 48, 48)], zbuf)
        pltpu.sync_copy(zbuf, out_hbm.at[pl.ds(c * N + r0 + t * 48, 48)])

    @pl.when(s == NS - 1)
    def _():
        pltpu.sync_copy(accum.at[pl.ds(NS * ROWS_A, 16)], zbuf.at[pl.ds(0, 16)])
        pltpu.sync_copy(zbuf.at[pl.ds(0, 16)],
                        out_hbm.at[pl.ds(c * N + NS * ROWS_A, 16)])


def _tc1(deg2, x, W1p):
    def body(deg2_ref, x_ref, w1_ref, dinv_ref, y1_ref):
        deg = 1.0 + deg2_ref[0, :] + deg2_ref[1, :]
        dinv = lax.rsqrt(deg)
        dinv_ref[...] = dinv
        y1_ref[...] = jnp.dot(x_ref[...], w1_ref[...],
                              preferred_element_type=jnp.float32) * dinv[:, None]

    return pl.pallas_call(
        body,
        out_shape=(jax.ShapeDtypeStruct((N,), jnp.float32),
                   jax.ShapeDtypeStruct((N, 128), jnp.float32)),
    )(deg2, x, W1p)


def _tc2(agg1, y1, dinv, W2p, b1):
    H = b1.shape[0]

    def body(agg_ref, y1_ref, dinv_ref, w2p_ref, b1_ref, y2_ref):
        dinv = dinv_ref[...]
        h = ((agg_ref[0, :, :H] + agg_ref[1, :, :H] + y1_ref[:, :H])
             * dinv[:, None] + b1_ref[...])
        h = jnp.maximum(h, 0.0)
        y2_ref[...] = jnp.dot(h, w2p_ref[...],
                              preferred_element_type=jnp.float32) * dinv[:, None]

    return pl.pallas_call(
        body,
        out_shape=jax.ShapeDtypeStruct((N, 128), jnp.float32),
    )(agg1, y1, dinv, W2p, b1)


def _tc3(agg2, y2, dinv, b2):
    NCls = b2.shape[0]

    def body(agg_ref, y2_ref, dinv_ref, b2_ref, out_ref):
        o = ((agg_ref[0, :, :48] + agg_ref[1, :, :48] + y2_ref[:, :48])
             * dinv_ref[...][:, None])
        o = o[:, :NCls] + b2_ref[...]
        m = jnp.max(o, axis=1, keepdims=True)
        lse = jnp.log(jnp.sum(jnp.exp(o - m), axis=1, keepdims=True)) + m
        out_ref[...] = o - lse

    return pl.pallas_call(
        body,
        out_shape=jax.ShapeDtypeStruct((N, NCls), jnp.float32),
    )(agg2, y2, dinv, b2)


_edge64 = _make_edge(64)
_edge48 = _make_edge(48)


def kernel(x, edge_index, edge_weight, W1, b1, W2, b2):
    src = edge_index[0].astype(jnp.int32)
    dst = edge_index[1].astype(jnp.int32)
    ew = edge_weight.astype(jnp.float32)
    W1p = jnp.pad(W1, ((0, 0), (0, 128 - W1.shape[1])))
    W2p = jnp.pad(W2, ((0, 0), (0, 128 - W2.shape[1])))

    deg2 = _deg_kernel(dst, ew).reshape(NC, N)
    dinv, y1 = _tc1(deg2, x, W1p)
    agg1 = _edge64(y1, src, dst, ew)
    y2 = _tc2(agg1, y1, dinv, W2p, b1)
    agg2 = _edge48(y2, src, dst, ew)
    return _tc3(agg2, y2, dinv, b2)


# speedup vs baseline: 20.3913x; 1.9794x over previous
import functools

import jax
import jax.numpy as jnp
from jax import lax
from jax.experimental import pallas as pl
from jax.experimental.pallas import tpu as pltpu
from jax.experimental.pallas import tpu_sc as plsc

N = 10000
E = 320000
NC = 2
NS = 16
NW = NC * NS
CH = 128
CPT = 80
E2 = NW * CPT * CH
ROWS_A = 624
EXB = 104

_MESH = plsc.VectorSubcoreMesh(core_axis_name="c", subcore_axis_name="s")


def _make_edge(D):
    SL = D // 16

    @functools.partial(
        pl.kernel,
        out_type=jax.ShapeDtypeStruct((NC, N, 128), jnp.float32),
        mesh=_MESH,
        scratch_types=[
            pltpu.VMEM((CPT // 2, CH), jnp.int32),
            pltpu.VMEM((CPT // 2, CH), jnp.int32),
            pltpu.VMEM((CPT // 2, CH), jnp.float32),
            pltpu.VMEM((CH, 128), jnp.float32),
            pltpu.VMEM((CH, 128), jnp.float32),
            pltpu.VMEM_SHARED((N, 128), jnp.float32),
            pltpu.SemaphoreType.DMA,
            pltpu.SemaphoreType.DMA,
            pltpu.SemaphoreType.DMA,
            pltpu.SemaphoreType.DMA,
        ],
    )
    def k(y_hbm, src_hbm, dst_hbm, ew_hbm, out_hbm,
          sidx, didx, ewv, rg0, rg1, accum,
          sg0, sg1, ss0, ss1):
        c = lax.axis_index("c")
        s = lax.axis_index("s")
        wid = c * NS + s
        r0 = s * ROWS_A
        HC = CPT // 2

        def rg0_zero(i, carry):
            for j in range(8):
                rg0[i, pl.ds(j * 16, 16)] = jnp.zeros((16,), jnp.float32)
            return carry

        lax.fori_loop(0, EXB, rg0_zero, 0)
        for t in range(ROWS_A // EXB):
            pltpu.sync_copy(rg0.at[pl.ds(0, EXB)],
                            accum.at[pl.ds(r0 + t * EXB, EXB)])

        @pl.when(s == NS - 1)
        def _():
            pltpu.sync_copy(rg0.at[pl.ds(0, 16)],
                            accum.at[pl.ds(NS * ROWS_A, 16)])

        plsc.subcore_barrier()

        def scale(jj, rg):
            def group_body(g, carry):
                wv = ewv[jj, pl.ds(g * 16, 16)]
                for i in range(16):
                    w = wv[i]
                    e = g * 16 + i
                    for j2 in range(SL):
                        rg[e, pl.ds(j2 * 16, 16)] = (
                            rg[e, pl.ds(j2 * 16, 16)] * w)
                return carry

            lax.fori_loop(0, CH // 16, group_body, 0)

        for h in range(2):
            base = wid * CPT + h * HC
            pltpu.sync_copy(src_hbm.at[pl.ds(base, HC)], sidx)
            pltpu.sync_copy(dst_hbm.at[pl.ds(base, HC)], didx)
            pltpu.sync_copy(ew_hbm.at[pl.ds(base, HC)], ewv)

            def pipe(i, carry):
                ja = 2 * i
                jb = 2 * i + 1
                da = pltpu.async_copy(y_hbm.at[sidx.at[ja]], rg0, sg0)
                da.wait()
                scale(ja, rg0)
                sa = pltpu.async_copy(rg0, accum.at[didx.at[ja]], ss0,
                                      add=True)
                sa.wait()
                db = pltpu.async_copy(y_hbm.at[sidx.at[jb]], rg1, sg1)
                db.wait()
                scale(jb, rg1)
                sb = pltpu.async_copy(rg1, accum.at[didx.at[jb]], ss1,
                                      add=True)
                sb.wait()
                return carry

            lax.fori_loop(0, HC // 2, pipe, 0)
        plsc.subcore_barrier()

        for t in range(ROWS_A // EXB):
            pltpu.sync_copy(accum.at[pl.ds(r0 + t * EXB, EXB)],
                            rg0.at[pl.ds(0, EXB)])
            pltpu.sync_copy(rg0.at[pl.ds(0, EXB)],
                            out_hbm.at[c, pl.ds(r0 + t * EXB, EXB)])

        @pl.when(s == NS - 1)
        def _():
            pltpu.sync_copy(accum.at[pl.ds(NS * ROWS_A, 16)],
                            rg0.at[pl.ds(0, 16)])
            pltpu.sync_copy(rg0.at[pl.ds(0, 16)],
                            out_hbm.at[c, pl.ds(NS * ROWS_A, 16)])

    return k


@functools.partial(
    pl.kernel,
    out_type=jax.ShapeDtypeStruct((NC * N,), jnp.float32),
    mesh=_MESH,
    scratch_types=[
        pltpu.VMEM((CPT, CH), jnp.int32),
        pltpu.VMEM((CPT, CH), jnp.float32),
        pltpu.VMEM((48,), jnp.float32),
        pltpu.VMEM_SHARED((N,), jnp.float32),
        pltpu.SemaphoreType.DMA,
        pltpu.SemaphoreType.DMA,
    ],
)
def _deg_kernel(dst_hbm, ew_hbm, out_hbm, didx, ewv, zbuf, accum, sem, sem2):
    c = lax.axis_index("c")
    s = lax.axis_index("s")
    wid = c * NS + s
    r0 = s * ROWS_A
    pltpu.sync_copy(dst_hbm.at[pl.ds(wid * CPT, CPT)], didx)
    pltpu.sync_copy(ew_hbm.at[pl.ds(wid * CPT, CPT)], ewv)
    for j in range(3):
        zbuf[pl.ds(j * 16, 16)] = jnp.zeros((16,), jnp.float32)
    for t in range(ROWS_A // 48):
        pltpu.sync_copy(zbuf, accum.at[pl.ds(r0 + t * 48, 48)])

    @pl.when(s == NS - 1)
    def _():
        pltpu.sync_copy(zbuf.at[pl.ds(0, 16)], accum.at[pl.ds(NS * ROWS_A, 16)])

    plsc.subcore_barrier()

    def fire(j, carry):
        da = pltpu.async_copy(ewv.at[2 * j], accum.at[didx.at[2 * j]], sem,
                              add=True)
        db = pltpu.async_copy(ewv.at[2 * j + 1], accum.at[didx.at[2 * j + 1]],
                              sem2, add=True)
        da.wait()
        db.wait()
        return carry

    lax.fori_loop(0, CPT // 2, fire, 0)
    plsc.subcore_barrier()

    for t in range(ROWS_A // 48):
        pltpu.sync_copy(accum.at[pl.ds(r0 + t * 48, 48)], zbuf)
        pltpu.sync_copy(zbuf, out_hbm.at[pl.ds(c * N + r0 + t * 48, 48)])

    @pl.when(s == NS - 1)
    def _():
        pltpu.sync_copy(accum.at[pl.ds(NS * ROWS_A, 16)], zbuf.at[pl.ds(0, 16)])
        pltpu.sync_copy(zbuf.at[pl.ds(0, 16)],
                        out_hbm.at[pl.ds(c * N + NS * ROWS_A, 16)])


def _tc1(deg2, x, W1p):
    def body(deg2_ref, x_ref, w1_ref, dinv_ref, y1_ref):
        deg = 1.0 + deg2_ref[0, :] + deg2_ref[1, :]
        dinv = lax.rsqrt(deg)
        dinv_ref[...] = dinv
        y1_ref[...] = jnp.dot(x_ref[...], w1_ref[...],
                              preferred_element_type=jnp.float32) * dinv[:, None]

    return pl.pallas_call(
        body,
        out_shape=(jax.ShapeDtypeStruct((N,), jnp.float32),
                   jax.ShapeDtypeStruct((N, 128), jnp.float32)),
    )(deg2, x, W1p)


def _tc2(agg1, y1, dinv, W2p, b1):
    H = b1.shape[0]

    def body(agg_ref, y1_ref, dinv_ref, w2p_ref, b1_ref, y2_ref):
        dinv = dinv_ref[...]
        h = ((agg_ref[0, :, :H] + agg_ref[1, :, :H] + y1_ref[:, :H])
             * dinv[:, None] + b1_ref[...])
        h = jnp.maximum(h, 0.0)
        y2_ref[...] = jnp.dot(h, w2p_ref[...],
                              preferred_element_type=jnp.float32) * dinv[:, None]

    return pl.pallas_call(
        body,
        out_shape=jax.ShapeDtypeStruct((N, 128), jnp.float32),
    )(agg1, y1, dinv, W2p, b1)


def _tc3(agg2, y2, dinv, b2):
    NCls = b2.shape[0]

    def body(agg_ref, y2_ref, dinv_ref, b2_ref, out_ref):
        o = ((agg_ref[0, :, :48] + agg_ref[1, :, :48] + y2_ref[:, :48])
             * dinv_ref[...][:, None])
        o = o[:, :NCls] + b2_ref[...]
        m = jnp.max(o, axis=1, keepdims=True)
        lse = jnp.log(jnp.sum(jnp.exp(o - m), axis=1, keepdims=True)) + m
        out_ref[...] = o - lse

    return pl.pallas_call(
        body,
        out_shape=jax.ShapeDtypeStruct((N, NCls), jnp.float32),
    )(agg2, y2, dinv, b2)


_edge64 = _make_edge(64)
_edge48 = _make_edge(48)


def kernel(x, edge_index, edge_weight, W1, b1, W2, b2):
    src = edge_index[0].astype(jnp.int32)
    dst = edge_index[1].astype(jnp.int32)
    ew = edge_weight.astype(jnp.float32)
    npad = E2 - E
    pad_idx = (jnp.arange(npad, dtype=jnp.int32) * 61) % N
    src2 = jnp.concatenate([src, pad_idx]).reshape(NW * CPT, CH)
    dst2 = jnp.concatenate([dst, pad_idx]).reshape(NW * CPT, CH)
    ew2 = jnp.concatenate([ew, jnp.zeros((npad,), jnp.float32)]
                          ).reshape(NW * CPT, CH)
    W1p = jnp.pad(W1, ((0, 0), (0, 128 - W1.shape[1])))
    W2p = jnp.pad(W2, ((0, 0), (0, 128 - W2.shape[1])))

    deg2 = _deg_kernel(dst2, ew2).reshape(NC, N)
    dinv, y1 = _tc1(deg2, x, W1p)
    agg1 = _edge64(y1, src2, dst2, ew2)
    y2 = _tc2(agg1, y1, dinv, W2p, b1)
    agg2 = _edge48(y2, src2, dst2, ew2)
    return _tc3(agg2, y2, dinv, b2)


# speedup vs baseline: 24.0398x; 1.1789x over previous
import functools

import jax
import jax.numpy as jnp
from jax import lax
from jax.experimental import pallas as pl
from jax.experimental.pallas import tpu as pltpu
from jax.experimental.pallas import tpu_sc as plsc

N = 10000
E = 320000
NC = 2
NS = 16
NW = NC * NS
CH = 128
CPT = 80
E2 = NW * CPT * CH
ROWS_A = 624
EXB = 104

_MESH = plsc.VectorSubcoreMesh(core_axis_name="c", subcore_axis_name="s")


def _make_edge(D):
    SL = D // 16

    @functools.partial(
        pl.kernel,
        out_type=jax.ShapeDtypeStruct((NC, N, 128), jnp.float32),
        mesh=_MESH,
        scratch_types=[
            pltpu.VMEM((CPT // 2, CH), jnp.int32),
            pltpu.VMEM((CPT // 2, CH), jnp.int32),
            pltpu.VMEM((CPT // 2, CH), jnp.float32),
            pltpu.VMEM((CH, 128), jnp.float32),
            pltpu.VMEM((CH, 128), jnp.float32),
            pltpu.VMEM_SHARED((N, 128), jnp.float32),
            pltpu.SemaphoreType.DMA,
            pltpu.SemaphoreType.DMA,
            pltpu.SemaphoreType.DMA,
            pltpu.SemaphoreType.DMA,
        ],
    )
    def k(y_hbm, src_hbm, dst_hbm, ew_hbm, out_hbm,
          sidx, didx, ewv, rg0, rg1, accum,
          sg0, sg1, ss0, ss1):
        c = lax.axis_index("c")
        s = lax.axis_index("s")
        wid = c * NS + s
        r0 = s * ROWS_A
        HC = CPT // 2

        def rg0_zero(i, carry):
            for j in range(8):
                rg0[i, pl.ds(j * 16, 16)] = jnp.zeros((16,), jnp.float32)
            return carry

        lax.fori_loop(0, EXB, rg0_zero, 0)
        for t in range(ROWS_A // EXB):
            pltpu.sync_copy(rg0.at[pl.ds(0, EXB)],
                            accum.at[pl.ds(r0 + t * EXB, EXB)])

        @pl.when(s == NS - 1)
        def _():
            pltpu.sync_copy(rg0.at[pl.ds(0, 16)],
                            accum.at[pl.ds(NS * ROWS_A, 16)])

        plsc.subcore_barrier()

        def scale(jj, rg):
            def group_body(g, carry):
                wv = ewv[jj, pl.ds(g * 16, 16)]
                for i in range(16):
                    w = wv[i]
                    e = g * 16 + i
                    for j2 in range(SL):
                        rg[e, pl.ds(j2 * 16, 16)] = (
                            rg[e, pl.ds(j2 * 16, 16)] * w)
                return carry

            lax.fori_loop(0, CH // 16, group_body, 0)

        for h in range(2):
            base = wid * CPT + h * HC
            pltpu.sync_copy(src_hbm.at[pl.ds(base, HC)], sidx)
            pltpu.sync_copy(dst_hbm.at[pl.ds(base, HC)], didx)
            pltpu.sync_copy(ew_hbm.at[pl.ds(base, HC)], ewv)

            def pipe(i, carry):
                ja = 2 * i
                jb = 2 * i + 1
                da = pltpu.async_copy(y_hbm.at[sidx.at[ja]], rg0, sg0)
                db = pltpu.async_copy(y_hbm.at[sidx.at[jb]], rg1, sg1)
                da.wait()
                scale(ja, rg0)
                sa = pltpu.async_copy(rg0, accum.at[didx.at[ja]], ss0,
                                      add=True)
                db.wait()
                scale(jb, rg1)
                sb = pltpu.async_copy(rg1, accum.at[didx.at[jb]], ss1,
                                      add=True)
                sa.wait()
                sb.wait()
                return carry

            lax.fori_loop(0, HC // 2, pipe, 0)
        plsc.subcore_barrier()

        for t in range(ROWS_A // EXB):
            pltpu.sync_copy(accum.at[pl.ds(r0 + t * EXB, EXB)],
                            rg0.at[pl.ds(0, EXB)])
            pltpu.sync_copy(rg0.at[pl.ds(0, EXB)],
                            out_hbm.at[c, pl.ds(r0 + t * EXB, EXB)])

        @pl.when(s == NS - 1)
        def _():
            pltpu.sync_copy(accum.at[pl.ds(NS * ROWS_A, 16)],
                            rg0.at[pl.ds(0, 16)])
            pltpu.sync_copy(rg0.at[pl.ds(0, 16)],
                            out_hbm.at[c, pl.ds(NS * ROWS_A, 16)])

    return k


@functools.partial(
    pl.kernel,
    out_type=jax.ShapeDtypeStruct((NC * N,), jnp.float32),
    mesh=_MESH,
    scratch_types=[
        pltpu.VMEM((CPT, CH), jnp.int32),
        pltpu.VMEM((CPT, CH), jnp.float32),
        pltpu.VMEM((48,), jnp.float32),
        pltpu.VMEM_SHARED((N,), jnp.float32),
        pltpu.SemaphoreType.DMA,
        pltpu.SemaphoreType.DMA,
    ],
)
def _deg_kernel(dst_hbm, ew_hbm, out_hbm, didx, ewv, zbuf, accum, sem, sem2):
    c = lax.axis_index("c")
    s = lax.axis_index("s")
    wid = c * NS + s
    r0 = s * ROWS_A
    pltpu.sync_copy(dst_hbm.at[pl.ds(wid * CPT, CPT)], didx)
    pltpu.sync_copy(ew_hbm.at[pl.ds(wid * CPT, CPT)], ewv)
    for j in range(3):
        zbuf[pl.ds(j * 16, 16)] = jnp.zeros((16,), jnp.float32)
    for t in range(ROWS_A // 48):
        pltpu.sync_copy(zbuf, accum.at[pl.ds(r0 + t * 48, 48)])

    @pl.when(s == NS - 1)
    def _():
        pltpu.sync_copy(zbuf.at[pl.ds(0, 16)], accum.at[pl.ds(NS * ROWS_A, 16)])

    plsc.subcore_barrier()

    def fire(j, carry):
        da = pltpu.async_copy(ewv.at[2 * j], accum.at[didx.at[2 * j]], sem,
                              add=True)
        db = pltpu.async_copy(ewv.at[2 * j + 1], accum.at[didx.at[2 * j + 1]],
                              sem2, add=True)
        da.wait()
        db.wait()
        return carry

    lax.fori_loop(0, CPT // 2, fire, 0)
    plsc.subcore_barrier()

    for t in range(ROWS_A // 48):
        pltpu.sync_copy(accum.at[pl.ds(r0 + t * 48, 48)], zbuf)
        pltpu.sync_copy(zbuf, out_hbm.at[pl.ds(c * N + r0 + t * 48, 48)])

    @pl.when(s == NS - 1)
    def _():
        pltpu.sync_copy(accum.at[pl.ds(NS * ROWS_A, 16)], zbuf.at[pl.ds(0, 16)])
        pltpu.sync_copy(zbuf.at[pl.ds(0, 16)],
                        out_hbm.at[pl.ds(c * N + NS * ROWS_A, 16)])


def _tc1(deg2, x, W1p):
    def body(deg2_ref, x_ref, w1_ref, dinv_ref, y1_ref):
        deg = 1.0 + deg2_ref[0, :] + deg2_ref[1, :]
        dinv = lax.rsqrt(deg)
        dinv_ref[...] = dinv
        y1_ref[...] = jnp.dot(x_ref[...], w1_ref[...],
                              preferred_element_type=jnp.float32) * dinv[:, None]

    return pl.pallas_call(
        body,
        out_shape=(jax.ShapeDtypeStruct((N,), jnp.float32),
                   jax.ShapeDtypeStruct((N, 128), jnp.float32)),
    )(deg2, x, W1p)


def _tc2(agg1, y1, dinv, W2p, b1):
    H = b1.shape[0]

    def body(agg_ref, y1_ref, dinv_ref, w2p_ref, b1_ref, y2_ref):
        dinv = dinv_ref[...]
        h = ((agg_ref[0, :, :H] + agg_ref[1, :, :H] + y1_ref[:, :H])
             * dinv[:, None] + b1_ref[...])
        h = jnp.maximum(h, 0.0)
        y2_ref[...] = jnp.dot(h, w2p_ref[...],
                              preferred_element_type=jnp.float32) * dinv[:, None]

    return pl.pallas_call(
        body,
        out_shape=jax.ShapeDtypeStruct((N, 128), jnp.float32),
    )(agg1, y1, dinv, W2p, b1)


def _tc3(agg2, y2, dinv, b2):
    NCls = b2.shape[0]

    def body(agg_ref, y2_ref, dinv_ref, b2_ref, out_ref):
        o = ((agg_ref[0, :, :48] + agg_ref[1, :, :48] + y2_ref[:, :48])
             * dinv_ref[...][:, None])
        o = o[:, :NCls] + b2_ref[...]
        m = jnp.max(o, axis=1, keepdims=True)
        lse = jnp.log(jnp.sum(jnp.exp(o - m), axis=1, keepdims=True)) + m
        out_ref[...] = o - lse

    return pl.pallas_call(
        body,
        out_shape=jax.ShapeDtypeStruct((N, NCls), jnp.float32),
    )(agg2, y2, dinv, b2)


_edge64 = _make_edge(64)
_edge48 = _make_edge(48)


def kernel(x, edge_index, edge_weight, W1, b1, W2, b2):
    src = edge_index[0].astype(jnp.int32)
    dst = edge_index[1].astype(jnp.int32)
    ew = edge_weight.astype(jnp.float32)
    npad = E2 - E
    pad_idx = (jnp.arange(npad, dtype=jnp.int32) * 61) % N
    src2 = jnp.concatenate([src, pad_idx]).reshape(NW * CPT, CH)
    dst2 = jnp.concatenate([dst, pad_idx]).reshape(NW * CPT, CH)
    ew2 = jnp.concatenate([ew, jnp.zeros((npad,), jnp.float32)]
                          ).reshape(NW * CPT, CH)
    W1p = jnp.pad(W1, ((0, 0), (0, 128 - W1.shape[1])))
    W2p = jnp.pad(W2, ((0, 0), (0, 128 - W2.shape[1])))

    deg2 = _deg_kernel(dst2, ew2).reshape(NC, N)
    dinv, y1 = _tc1(deg2, x, W1p)
    agg1 = _edge64(y1, src2, dst2, ew2)
    y2 = _tc2(agg1, y1, dinv, W2p, b1)
    agg2 = _edge48(y2, src2, dst2, ew2)
    return _tc3(agg2, y2, dinv, b2)
